# baseline (speedup 1.0000x reference)
import jax
import jax.numpy as jnp
from jax.experimental import pallas as pl
from jax.experimental.pallas import tpu as pltpu

SOM_H, SOM_W, D = 64, 64, 512
HW = SOM_H * SOM_W
BATCH = 4096
TB = 1024
NT = BATCH // TB


def _som_kernel(x_ref, w_ref, coord_ref, idx_ref,
                wsq_ref, crA, crB, xsA, xsB):
    i = pl.program_id(0)

    @pl.when(i == 0)
    def _():
        w = w_ref[...]
        wsq_ref[...] = jnp.sum(w * w, axis=1)[None, :]

    x = x_ref[...]
    xn = -2.0 * x
    x_sq = jnp.sum(x * x, axis=1, keepdims=True)

    def stage(cr_w, xs_w):
        xs_w[...] = x_sq
        cr_w[...] = jax.lax.dot_general(
            xn, w_ref[...], (((1,), (1,)), ((), ())),
            preferred_element_type=jnp.float32,
        )

    def epilogue(cr_r, xs_r):
        dist = (xs_r[...] + cr_r[...]) + wsq_ref[...]
        idx = jnp.argmin(dist, axis=1).astype(jnp.int32)
        idx_ref[...] = idx[:, None]
        coord_ref[...] = jnp.stack([idx // SOM_W, idx % SOM_W], axis=1)

    even = jax.lax.rem(i, 2) == 0

    @pl.when(even)
    def _():
        stage(crA, xsA)
        epilogue(crB, xsB)

    @pl.when(jnp.logical_not(even))
    def _():
        stage(crB, xsB)
        epilogue(crA, xsA)


def kernel(x, weights):
    wf = weights.reshape(HW, D)
    grid = (NT + 1,)
    coords, idx = pl.pallas_call(
        _som_kernel,
        grid=grid,
        in_specs=[
            pl.BlockSpec((TB, D), lambda i: (jnp.minimum(i, NT - 1), 0)),
            pl.BlockSpec((HW, D), lambda i: (0, 0)),
        ],
        out_specs=[
            pl.BlockSpec((TB, 2), lambda i: (jnp.maximum(i - 1, 0), 0)),
            pl.BlockSpec((TB, 1), lambda i: (jnp.maximum(i - 1, 0), 0)),
        ],
        out_shape=[
            jax.ShapeDtypeStruct((BATCH, 2), jnp.int32),
            jax.ShapeDtypeStruct((BATCH, 1), jnp.int32),
        ],
        scratch_shapes=[
            pltpu.VMEM((1, HW), jnp.float32),
            pltpu.VMEM((TB, HW), jnp.float32),
            pltpu.VMEM((TB, HW), jnp.float32),
            pltpu.VMEM((TB, 1), jnp.float32),
            pltpu.VMEM((TB, 1), jnp.float32),
        ],
    )(x, wf)
    return coords, idx[:, 0]


# speedup vs baseline: 1.5509x; 1.5509x over previous
import jax
import jax.numpy as jnp
from jax.experimental import pallas as pl
from jax.experimental.pallas import tpu as pltpu

SOM_H, SOM_W, D = 64, 64, 512
HW = SOM_H * SOM_W
BATCH = 4096
TB = 1024


def _som_kernel(x_ref, w_ref, coord_ref, idx_ref, wsq_ref):
    @pl.when(pl.program_id(0) == 0)
    def _():
        w = w_ref[...]
        wsq_ref[...] = jnp.sum(w * w, axis=1)[None, :]

    x = x_ref[...]
    xn = -2.0 * x
    x_sq = jnp.sum(x * x, axis=1, keepdims=True)
    cross2 = jax.lax.dot_general(
        xn, w_ref[...], (((1,), (1,)), ((), ())),
        preferred_element_type=jnp.float32,
    )
    dist = (x_sq + cross2) + wsq_ref[...]
    idx = jnp.argmin(dist, axis=1).astype(jnp.int32)
    idx_ref[...] = idx[:, None]
    coord_ref[...] = jnp.stack([idx // SOM_W, idx % SOM_W], axis=1)


def kernel(x, weights):
    wneg = weights.reshape(HW, D)
    grid = (BATCH // TB,)
    coords, idx = pl.pallas_call(
        _som_kernel,
        grid=grid,
        in_specs=[
            pl.BlockSpec((TB, D), lambda i: (i, 0)),
            pl.BlockSpec((HW, D), lambda i: (0, 0)),
        ],
        out_specs=[
            pl.BlockSpec((TB, 2), lambda i: (i, 0)),
            pl.BlockSpec((TB, 1), lambda i: (i, 0)),
        ],
        out_shape=[
            jax.ShapeDtypeStruct((BATCH, 2), jnp.int32),
            jax.ShapeDtypeStruct((BATCH, 1), jnp.int32),
        ],
        scratch_shapes=[pltpu.VMEM((1, HW), jnp.float32)],
    )(x, wneg)
    return coords, idx[:, 0]


# speedup vs baseline: 1.5639x; 1.0084x over previous
import jax
import jax.numpy as jnp
from jax.experimental import pallas as pl
from jax.experimental.pallas import tpu as pltpu

SOM_H, SOM_W, D = 64, 64, 512
HW = SOM_H * SOM_W
BATCH = 4096
TB = 1024


def _som_kernel(x_ref, w_ref, coord_ref, idx_ref, wsq_ref):
    @pl.when(pl.program_id(0) == 0)
    def _():
        w = w_ref[...]
        wsq_ref[...] = jnp.sum(w * w, axis=1)[None, :]

    x = x_ref[...]
    xn = -2.0 * x
    x_sq = jnp.sum(x * x, axis=1, keepdims=True)
    xnt = xn.T
    cross2 = jax.lax.dot_general(
        xnt, w_ref[...], (((0,), (1,)), ((), ())),
        preferred_element_type=jnp.float32,
    )
    dist = (x_sq + cross2) + wsq_ref[...]
    idx = jnp.argmin(dist, axis=1).astype(jnp.int32)
    idx_ref[...] = idx[:, None]
    coord_ref[...] = jnp.stack([idx // SOM_W, idx % SOM_W], axis=1)


def kernel(x, weights):
    wneg = weights.reshape(HW, D)
    grid = (BATCH // TB,)
    coords, idx = pl.pallas_call(
        _som_kernel,
        grid=grid,
        in_specs=[
            pl.BlockSpec((TB, D), lambda i: (i, 0)),
            pl.BlockSpec((HW, D), lambda i: (0, 0)),
        ],
        out_specs=[
            pl.BlockSpec((TB, 2), lambda i: (i, 0)),
            pl.BlockSpec((TB, 1), lambda i: (i, 0)),
        ],
        out_shape=[
            jax.ShapeDtypeStruct((BATCH, 2), jnp.int32),
            jax.ShapeDtypeStruct((BATCH, 1), jnp.int32),
        ],
        scratch_shapes=[pltpu.VMEM((1, HW), jnp.float32)],
    )(x, wneg)
    return coords, idx[:, 0]
